# baseline (speedup 1.0000x reference)
import functools

import jax
import jax.numpy as jnp
from jax.experimental import pallas as pl
from jax.experimental.pallas import tpu as pltpu

B = 16
L = 2048
U = 1024
E = 128
V = 100000

_FC_BLK = 4096


def _attn_kernel(hidden_ref, enc_ref, w1_ref, w2_ref, v_ref, b12_ref,
                 aw_ref, ctx_ref, pre_ref, w1bf_ref):
    b = pl.program_id(0)

    @pl.when(b == 0)
    def _init():
        w1bf_ref[...] = w1_ref[...].astype(jnp.bfloat16)
        hbf = hidden_ref[...].astype(jnp.bfloat16)
        w2bf = w2_ref[...].astype(jnp.bfloat16)
        pre_ref[...] = (
            jnp.dot(hbf, w2bf, preferred_element_type=jnp.float32)
            + b12_ref[...]
        )

    encbf = enc_ref[0].astype(jnp.bfloat16)
    acc = jnp.dot(encbf, w1bf_ref[...], preferred_element_type=jnp.float32)
    tbf = jnp.tanh(acc + pre_ref[pl.ds(b, 1), :]).astype(jnp.bfloat16)
    s = jnp.dot(tbf, v_ref[...].astype(jnp.bfloat16),
                preferred_element_type=jnp.float32)
    sr = jnp.transpose(s)
    m = jnp.max(sr, axis=1, keepdims=True)
    e = jnp.exp(sr - m)
    w = e / jnp.sum(e, axis=1, keepdims=True)
    aw_ref[...] = w.reshape(1, 1, L)
    ctx_ref[...] = jnp.dot(w.astype(jnp.bfloat16), encbf,
                           preferred_element_type=jnp.float32).reshape(1, 1, U)


def _gru_kernel(x_ref, ctx_ref, hidden_ref, wk_ref, wrk_ref, gb_ref, grb_ref,
                emb_ref, state_ref, xe_ref, sem):
    for i in range(B):
        pltpu.make_async_copy(emb_ref.at[x_ref[i]], xe_ref.at[i], sem).start()
    for i in range(B):
        pltpu.make_async_copy(emb_ref.at[x_ref[i]], xe_ref.at[i], sem).wait()

    ctx_bf = ctx_ref[...].astype(jnp.bfloat16)
    xe_bf = xe_ref[...].astype(jnp.bfloat16)
    h_bf = hidden_ref[...].astype(jnp.bfloat16)
    wk_bf = wk_ref[...].astype(jnp.bfloat16)
    wrk_bf = wrk_ref[...].astype(jnp.bfloat16)
    mx = (jnp.dot(ctx_bf, wk_bf[:U, :], preferred_element_type=jnp.float32)
          + jnp.dot(xe_bf, wk_bf[U:, :], preferred_element_type=jnp.float32)
          + gb_ref[...])
    mi = (jnp.dot(h_bf, wrk_bf, preferred_element_type=jnp.float32)
          + grb_ref[...])
    xz, xr, xh = mx[:, :U], mx[:, U:2 * U], mx[:, 2 * U:]
    rz, rr, rh = mi[:, :U], mi[:, U:2 * U], mi[:, 2 * U:]
    z = jax.nn.sigmoid(xz + rz)
    r = jax.nn.sigmoid(xr + rr)
    hh = jnp.tanh(xh + r * rh)
    state_ref[...] = z * hidden_ref[...] + (1.0 - z) * hh


def _fc_kernel(state_ref, wt_ref, b_ref, out_ref):
    sbf = state_ref[...].astype(jnp.bfloat16)
    wtbf = wt_ref[...].astype(jnp.bfloat16)
    acc = jax.lax.dot_general(sbf, wtbf, (((1,), (1,)), ((), ())),
                              preferred_element_type=jnp.float32)
    out_ref[...] = acc + b_ref[...]


@functools.partial(jax.jit, static_argnames=())
def kernel(x, hidden, enc_output, emb, att_W1, att_b1, att_W2, att_b2, att_V,
           att_Vb, gru_kernel, gru_rkernel, gru_bias, gru_rbias, fc_W, fc_b):
    bias12 = (att_b1 + att_b2).reshape(1, U)

    aw, ctx = pl.pallas_call(
        _attn_kernel,
        grid=(B,),
        in_specs=[
            pl.BlockSpec((B, U), lambda b: (0, 0)),
            pl.BlockSpec((1, L, U), lambda b: (b, 0, 0)),
            pl.BlockSpec((U, U), lambda b: (0, 0)),
            pl.BlockSpec((U, U), lambda b: (0, 0)),
            pl.BlockSpec((U, 1), lambda b: (0, 0)),
            pl.BlockSpec((1, U), lambda b: (0, 0)),
        ],
        out_specs=[
            pl.BlockSpec((1, 1, L), lambda b: (b, 0, 0)),
            pl.BlockSpec((1, 1, U), lambda b: (b, 0, 0)),
        ],
        out_shape=[
            jax.ShapeDtypeStruct((B, 1, L), jnp.float32),
            jax.ShapeDtypeStruct((B, 1, U), jnp.float32),
        ],
        scratch_shapes=[
            pltpu.VMEM((B, U), jnp.float32),
            pltpu.VMEM((U, U), jnp.bfloat16),
        ],
    )(hidden, enc_output, att_W1, att_W2, att_V, bias12)
    ctx = ctx.reshape(B, U)
    aw = jnp.transpose(aw, (0, 2, 1))

    state = pl.pallas_call(
        _gru_kernel,
        in_specs=[
            pl.BlockSpec(memory_space=pltpu.SMEM),
            pl.BlockSpec((B, U), lambda: (0, 0)),
            pl.BlockSpec((B, U), lambda: (0, 0)),
            pl.BlockSpec((U + E, 3 * U), lambda: (0, 0)),
            pl.BlockSpec((U, 3 * U), lambda: (0, 0)),
            pl.BlockSpec((1, 3 * U), lambda: (0, 0)),
            pl.BlockSpec((1, 3 * U), lambda: (0, 0)),
            pl.BlockSpec(memory_space=pl.ANY),
        ],
        out_specs=pl.BlockSpec((B, U), lambda: (0, 0)),
        out_shape=jax.ShapeDtypeStruct((B, U), jnp.float32),
        scratch_shapes=[
            pltpu.VMEM((B, E), jnp.float32),
            pltpu.SemaphoreType.DMA,
        ],
    )(x.reshape(B), ctx, hidden, gru_kernel, gru_rkernel,
      gru_bias.reshape(1, 3 * U), gru_rbias.reshape(1, 3 * U), emb)

    n_blk = pl.cdiv(V, _FC_BLK)
    logits = pl.pallas_call(
        _fc_kernel,
        grid=(n_blk,),
        in_specs=[
            pl.BlockSpec((B, U), lambda i: (0, 0)),
            pl.BlockSpec((_FC_BLK, U), lambda i: (i, 0)),
            pl.BlockSpec((1, _FC_BLK), lambda i: (0, i)),
        ],
        out_specs=pl.BlockSpec((B, _FC_BLK), lambda i: (0, i)),
        out_shape=jax.ShapeDtypeStruct((B, V), jnp.float32),
    )(state, fc_W.T, fc_b.reshape(1, V))

    return (logits, state, aw)


# speedup vs baseline: 1.0461x; 1.0461x over previous
import functools

import jax
import jax.numpy as jnp
from jax.experimental import pallas as pl
from jax.experimental.pallas import tpu as pltpu

B = 16
L = 2048
U = 1024
E = 128
V = 100000

_FC_BLK = 4096


_LC = 512


def _prep_kernel(hidden_ref, w1_ref, w2_ref, b12_ref, pre_ref, w1bf_ref):
    w1bf_ref[...] = w1_ref[...].astype(jnp.bfloat16)
    hbf = hidden_ref[...].astype(jnp.bfloat16)
    w2bf = w2_ref[...].astype(jnp.bfloat16)
    pre_ref[...] = (
        jnp.dot(hbf, w2bf, preferred_element_type=jnp.float32) + b12_ref[...]
    )


def _attn_kernel(pre_ref, enc_ref, w1bf_ref, v_ref, aw_ref, ctx_ref):
    b = pl.program_id(0)
    pre_b = pre_ref[pl.ds(b, 1), :]
    vbf = v_ref[...].astype(jnp.bfloat16)
    w1bf = w1bf_ref[...]
    enc_chunks = []
    s_chunks = []
    for c in range(L // _LC):
        encbf = enc_ref[0, pl.ds(c * _LC, _LC), :].astype(jnp.bfloat16)
        acc = jnp.dot(encbf, w1bf, preferred_element_type=jnp.float32)
        tbf = jnp.tanh(acc + pre_b).astype(jnp.bfloat16)
        sc = jnp.dot(tbf, vbf, preferred_element_type=jnp.float32)
        enc_chunks.append(encbf)
        s_chunks.append(jnp.transpose(sc))
    sr = jnp.concatenate(s_chunks, axis=1)
    m = jnp.max(sr, axis=1, keepdims=True)
    e = jnp.exp(sr - m)
    w = e / jnp.sum(e, axis=1, keepdims=True)
    aw_ref[...] = w.reshape(1, 1, L)
    encbf_all = jnp.concatenate(enc_chunks, axis=0)
    ctx_ref[...] = jnp.dot(w.astype(jnp.bfloat16), encbf_all,
                           preferred_element_type=jnp.float32).reshape(1, 1, U)


def _gru_kernel(x_ref, ctx_ref, hidden_ref, wk_ref, wrk_ref, gb_ref, grb_ref,
                emb_ref, state_ref, xe_ref, sem):
    for i in range(B):
        pltpu.make_async_copy(emb_ref.at[x_ref[i]], xe_ref.at[i], sem).start()
    for i in range(B):
        pltpu.make_async_copy(emb_ref.at[x_ref[i]], xe_ref.at[i], sem).wait()

    ctx_bf = ctx_ref[...].astype(jnp.bfloat16)
    xe_bf = xe_ref[...].astype(jnp.bfloat16)
    h_bf = hidden_ref[...].astype(jnp.bfloat16)
    wk_bf = wk_ref[...].astype(jnp.bfloat16)
    wrk_bf = wrk_ref[...].astype(jnp.bfloat16)
    mx = (jnp.dot(ctx_bf, wk_bf[:U, :], preferred_element_type=jnp.float32)
          + jnp.dot(xe_bf, wk_bf[U:, :], preferred_element_type=jnp.float32)
          + gb_ref[...])
    mi = (jnp.dot(h_bf, wrk_bf, preferred_element_type=jnp.float32)
          + grb_ref[...])
    xz, xr, xh = mx[:, :U], mx[:, U:2 * U], mx[:, 2 * U:]
    rz, rr, rh = mi[:, :U], mi[:, U:2 * U], mi[:, 2 * U:]
    z = jax.nn.sigmoid(xz + rz)
    r = jax.nn.sigmoid(xr + rr)
    hh = jnp.tanh(xh + r * rh)
    state_ref[...] = z * hidden_ref[...] + (1.0 - z) * hh


def _fc_kernel(state_ref, wt_ref, b_ref, out_ref):
    sbf = state_ref[...].astype(jnp.bfloat16)
    wtbf = wt_ref[...].astype(jnp.bfloat16)
    acc = jax.lax.dot_general(sbf, wtbf, (((1,), (1,)), ((), ())),
                              preferred_element_type=jnp.float32)
    out_ref[...] = acc + b_ref[...]


@functools.partial(jax.jit, static_argnames=())
def kernel(x, hidden, enc_output, emb, att_W1, att_b1, att_W2, att_b2, att_V,
           att_Vb, gru_kernel, gru_rkernel, gru_bias, gru_rbias, fc_W, fc_b):
    bias12 = (att_b1 + att_b2).reshape(1, U)

    pre, w1bf = pl.pallas_call(
        _prep_kernel,
        in_specs=[
            pl.BlockSpec((B, U), lambda: (0, 0)),
            pl.BlockSpec((U, U), lambda: (0, 0)),
            pl.BlockSpec((U, U), lambda: (0, 0)),
            pl.BlockSpec((1, U), lambda: (0, 0)),
        ],
        out_specs=[
            pl.BlockSpec((B, U), lambda: (0, 0)),
            pl.BlockSpec((U, U), lambda: (0, 0)),
        ],
        out_shape=[
            jax.ShapeDtypeStruct((B, U), jnp.float32),
            jax.ShapeDtypeStruct((U, U), jnp.bfloat16),
        ],
    )(hidden, att_W1, att_W2, bias12)

    aw, ctx = pl.pallas_call(
        _attn_kernel,
        grid=(B,),
        in_specs=[
            pl.BlockSpec((B, U), lambda b: (0, 0)),
            pl.BlockSpec((1, L, U), lambda b: (b, 0, 0)),
            pl.BlockSpec((U, U), lambda b: (0, 0)),
            pl.BlockSpec((U, 1), lambda b: (0, 0)),
        ],
        out_specs=[
            pl.BlockSpec((1, 1, L), lambda b: (b, 0, 0)),
            pl.BlockSpec((1, 1, U), lambda b: (b, 0, 0)),
        ],
        out_shape=[
            jax.ShapeDtypeStruct((B, 1, L), jnp.float32),
            jax.ShapeDtypeStruct((B, 1, U), jnp.float32),
        ],
    )(pre, enc_output, w1bf, att_V)
    ctx = ctx.reshape(B, U)
    aw = jnp.transpose(aw, (0, 2, 1))

    state = pl.pallas_call(
        _gru_kernel,
        in_specs=[
            pl.BlockSpec(memory_space=pltpu.SMEM),
            pl.BlockSpec((B, U), lambda: (0, 0)),
            pl.BlockSpec((B, U), lambda: (0, 0)),
            pl.BlockSpec((U + E, 3 * U), lambda: (0, 0)),
            pl.BlockSpec((U, 3 * U), lambda: (0, 0)),
            pl.BlockSpec((1, 3 * U), lambda: (0, 0)),
            pl.BlockSpec((1, 3 * U), lambda: (0, 0)),
            pl.BlockSpec(memory_space=pl.ANY),
        ],
        out_specs=pl.BlockSpec((B, U), lambda: (0, 0)),
        out_shape=jax.ShapeDtypeStruct((B, U), jnp.float32),
        scratch_shapes=[
            pltpu.VMEM((B, E), jnp.float32),
            pltpu.SemaphoreType.DMA,
        ],
    )(x.reshape(B), ctx, hidden, gru_kernel, gru_rkernel,
      gru_bias.reshape(1, 3 * U), gru_rbias.reshape(1, 3 * U), emb)

    n_blk = pl.cdiv(V, _FC_BLK)
    logits = pl.pallas_call(
        _fc_kernel,
        grid=(n_blk,),
        in_specs=[
            pl.BlockSpec((B, U), lambda i: (0, 0)),
            pl.BlockSpec((_FC_BLK, U), lambda i: (i, 0)),
            pl.BlockSpec((1, _FC_BLK), lambda i: (0, i)),
        ],
        out_specs=pl.BlockSpec((B, _FC_BLK), lambda i: (0, i)),
        out_shape=jax.ShapeDtypeStruct((B, V), jnp.float32),
    )(state, fc_W.T, fc_b.reshape(1, V))

    return (logits, state, aw)


# speedup vs baseline: 1.0722x; 1.0250x over previous
import functools

import jax
import jax.numpy as jnp
from jax.experimental import pallas as pl
from jax.experimental.pallas import tpu as pltpu

B = 16
L = 2048
U = 1024
E = 128
V = 100000

_FC_BLK = 5120


_LC = 512


def _prep_kernel(hidden_ref, w1_ref, w2_ref, b12_ref, pre_ref, w1bf_ref):
    w1bf_ref[...] = w1_ref[...].astype(jnp.bfloat16)
    hbf = hidden_ref[...].astype(jnp.bfloat16)
    w2bf = w2_ref[...].astype(jnp.bfloat16)
    pre_ref[...] = (
        jnp.dot(hbf, w2bf, preferred_element_type=jnp.float32) + b12_ref[...]
    )


_BB = 2


def _attn_kernel(pre_ref, enc_ref, w1bf_ref, v_ref, aw_ref, ctx_ref):
    g = pl.program_id(0)
    vbf = v_ref[...].astype(jnp.bfloat16)
    w1bf = w1bf_ref[...]
    enc_all = []
    s_rows = []
    for i in range(_BB):
        pre_b = pre_ref[pl.ds(g * _BB + i, 1), :]
        enc_chunks = []
        s_chunks = []
        for c in range(L // _LC):
            encbf = enc_ref[i, pl.ds(c * _LC, _LC), :].astype(jnp.bfloat16)
            acc = jnp.dot(encbf, w1bf, preferred_element_type=jnp.float32)
            tbf = jnp.tanh(acc + pre_b).astype(jnp.bfloat16)
            sc = jnp.dot(tbf, vbf, preferred_element_type=jnp.float32)
            enc_chunks.append(encbf)
            s_chunks.append(jnp.transpose(sc))
        enc_all.append(jnp.concatenate(enc_chunks, axis=0))
        s_rows.append(jnp.concatenate(s_chunks, axis=1))
    for i in range(_BB):
        sr = s_rows[i]
        m = jnp.max(sr, axis=1, keepdims=True)
        e = jnp.exp(sr - m)
        w = e / jnp.sum(e, axis=1, keepdims=True)
        aw_ref[pl.ds(i, 1)] = w.reshape(1, 1, L)
        ctx_ref[pl.ds(i, 1)] = jnp.dot(
            w.astype(jnp.bfloat16), enc_all[i],
            preferred_element_type=jnp.float32).reshape(1, 1, U)


def _gru_kernel(x_ref, ctx_ref, hidden_ref, wk_ref, wrk_ref, gb_ref, grb_ref,
                emb_ref, state_ref, xe_ref, sem):
    for i in range(B):
        pltpu.make_async_copy(emb_ref.at[x_ref[i]], xe_ref.at[i], sem).start()
    for i in range(B):
        pltpu.make_async_copy(emb_ref.at[x_ref[i]], xe_ref.at[i], sem).wait()

    ctx_bf = ctx_ref[...].astype(jnp.bfloat16)
    xe_bf = xe_ref[...].astype(jnp.bfloat16)
    h_bf = hidden_ref[...].astype(jnp.bfloat16)
    wk_bf = wk_ref[...].astype(jnp.bfloat16)
    wrk_bf = wrk_ref[...].astype(jnp.bfloat16)
    mx = (jnp.dot(ctx_bf, wk_bf[:U, :], preferred_element_type=jnp.float32)
          + jnp.dot(xe_bf, wk_bf[U:, :], preferred_element_type=jnp.float32)
          + gb_ref[...])
    mi = (jnp.dot(h_bf, wrk_bf, preferred_element_type=jnp.float32)
          + grb_ref[...])
    xz, xr, xh = mx[:, :U], mx[:, U:2 * U], mx[:, 2 * U:]
    rz, rr, rh = mi[:, :U], mi[:, U:2 * U], mi[:, 2 * U:]
    z = jax.nn.sigmoid(xz + rz)
    r = jax.nn.sigmoid(xr + rr)
    hh = jnp.tanh(xh + r * rh)
    state_ref[...] = z * hidden_ref[...] + (1.0 - z) * hh


def _fc_kernel(state_ref, wt_ref, b_ref, out_ref):
    sbf = state_ref[...].astype(jnp.bfloat16)
    wtbf = wt_ref[...].astype(jnp.bfloat16)
    acc = jax.lax.dot_general(sbf, wtbf, (((1,), (1,)), ((), ())),
                              preferred_element_type=jnp.float32)
    out_ref[...] = acc + b_ref[...]


@functools.partial(jax.jit, static_argnames=())
def kernel(x, hidden, enc_output, emb, att_W1, att_b1, att_W2, att_b2, att_V,
           att_Vb, gru_kernel, gru_rkernel, gru_bias, gru_rbias, fc_W, fc_b):
    bias12 = (att_b1 + att_b2).reshape(1, U)

    pre, w1bf = pl.pallas_call(
        _prep_kernel,
        in_specs=[
            pl.BlockSpec((B, U), lambda: (0, 0)),
            pl.BlockSpec((U, U), lambda: (0, 0)),
            pl.BlockSpec((U, U), lambda: (0, 0)),
            pl.BlockSpec((1, U), lambda: (0, 0)),
        ],
        out_specs=[
            pl.BlockSpec((B, U), lambda: (0, 0)),
            pl.BlockSpec((U, U), lambda: (0, 0)),
        ],
        out_shape=[
            jax.ShapeDtypeStruct((B, U), jnp.float32),
            jax.ShapeDtypeStruct((U, U), jnp.bfloat16),
        ],
    )(hidden, att_W1, att_W2, bias12)

    aw, ctx = pl.pallas_call(
        _attn_kernel,
        grid=(B // _BB,),
        in_specs=[
            pl.BlockSpec((B, U), lambda b: (0, 0)),
            pl.BlockSpec((_BB, L, U), lambda b: (b, 0, 0)),
            pl.BlockSpec((U, U), lambda b: (0, 0)),
            pl.BlockSpec((U, 1), lambda b: (0, 0)),
        ],
        out_specs=[
            pl.BlockSpec((_BB, 1, L), lambda b: (b, 0, 0)),
            pl.BlockSpec((_BB, 1, U), lambda b: (b, 0, 0)),
        ],
        out_shape=[
            jax.ShapeDtypeStruct((B, 1, L), jnp.float32),
            jax.ShapeDtypeStruct((B, 1, U), jnp.float32),
        ],
    )(pre, enc_output, w1bf, att_V)
    ctx = ctx.reshape(B, U)
    aw = jnp.transpose(aw, (0, 2, 1))

    state = pl.pallas_call(
        _gru_kernel,
        in_specs=[
            pl.BlockSpec(memory_space=pltpu.SMEM),
            pl.BlockSpec((B, U), lambda: (0, 0)),
            pl.BlockSpec((B, U), lambda: (0, 0)),
            pl.BlockSpec((U + E, 3 * U), lambda: (0, 0)),
            pl.BlockSpec((U, 3 * U), lambda: (0, 0)),
            pl.BlockSpec((1, 3 * U), lambda: (0, 0)),
            pl.BlockSpec((1, 3 * U), lambda: (0, 0)),
            pl.BlockSpec(memory_space=pl.ANY),
        ],
        out_specs=pl.BlockSpec((B, U), lambda: (0, 0)),
        out_shape=jax.ShapeDtypeStruct((B, U), jnp.float32),
        scratch_shapes=[
            pltpu.VMEM((B, E), jnp.float32),
            pltpu.SemaphoreType.DMA,
        ],
    )(x.reshape(B), ctx, hidden, gru_kernel, gru_rkernel,
      gru_bias.reshape(1, 3 * U), gru_rbias.reshape(1, 3 * U), emb)

    n_blk = pl.cdiv(V, _FC_BLK)
    logits = pl.pallas_call(
        _fc_kernel,
        grid=(n_blk,),
        in_specs=[
            pl.BlockSpec((B, U), lambda i: (0, 0)),
            pl.BlockSpec((_FC_BLK, U), lambda i: (i, 0)),
            pl.BlockSpec((1, _FC_BLK), lambda i: (0, i)),
        ],
        out_specs=pl.BlockSpec((B, _FC_BLK), lambda i: (0, i)),
        out_shape=jax.ShapeDtypeStruct((B, V), jnp.float32),
    )(state, fc_W.T, fc_b.reshape(1, V))

    return (logits, state, aw)
